# baseline (speedup 1.0000x reference)
import functools

import jax
import jax.numpy as jnp
import numpy as np
from jax import lax
from jax.experimental import pallas as pl
from jax.experimental.pallas import tpu as pltpu
from jax.experimental.pallas import tpu_sc as plsc

N = 10000
E = 320000
D_IN = 128
H = 12
F = 16
NC = 2
NS = 16
NW = NC * NS
CH = 128
NCHT = E // CH
MAXC = NCHT // NW + 1
KB = 8
PD = 4
NP = 10240
STRIPE = NP // NS
VR = NP // 8
XR = N // 8


def _sc_mesh():
    return plsc.VectorSubcoreMesh(core_axis_name="c", subcore_axis_name="s",
                                  num_cores=NC, num_subcores=NS)


_SC_PARAMS = pltpu.CompilerParams(use_tc_tiling_on_sc=False)


def _worker_span(wid):
    base = (wid * NCHT) // NW
    cnt = (((wid + 1) * NCHT) // NW) - base
    return base, cnt


def _deg_body(src_hbm, dst_hbm, zeros_hbm, out_hbm,
              src_v, dst_v, ones_v, dout_sh, din_sh, sem):
    cid = lax.axis_index("c")
    sid = lax.axis_index("s")
    wid = sid * NC + cid
    base, cnt = _worker_span(wid)
    zrows = zeros_hbm.at[pl.ds(sid * STRIPE, STRIPE)]
    stages = (
        (zrows, dout_sh.at[pl.ds(sid * STRIPE, STRIPE)]),
        (zrows, din_sh.at[pl.ds(sid * STRIPE, STRIPE)]),
        (src_hbm.at[pl.ds(base, MAXC)], src_v),
        (dst_hbm.at[pl.ds(base, MAXC)], dst_v),
    )
    for a, b_ in stages:
        pltpu.async_copy(a, b_, sem)
    for a, b_ in stages:
        pltpu.make_async_copy(a, b_, sem).wait()
    def fill(i, _):
        ones_v[i, :] = jnp.full((F,), 1.0, jnp.float32)
        return _
    lax.fori_loop(0, CH, fill, 0)
    plsc.subcore_barrier()
    def chunk(c, _):
        pltpu.async_copy(ones_v, dout_sh.at[src_v.at[c]], sem, add=True)
        pltpu.async_copy(ones_v, din_sh.at[dst_v.at[c]], sem, add=True)
        return _
    lax.fori_loop(0, cnt, chunk, 0)
    def drain(c, _):
        pltpu.make_async_copy(ones_v, dout_sh.at[src_v.at[0]], sem).wait()
        pltpu.make_async_copy(ones_v, din_sh.at[dst_v.at[0]], sem).wait()
        return _
    lax.fori_loop(0, cnt, drain, 0)
    plsc.subcore_barrier()
    s = pl.ds(sid * STRIPE, STRIPE)
    pltpu.sync_copy(dout_sh.at[s], out_hbm.at[cid, 0, s])
    pltpu.sync_copy(din_sh.at[s], out_hbm.at[cid, 1, s])


def _agg_body(table_hbm, src_hbm, dst_hbm, zeros_hbm, out_hbm,
              src_v, dst_v, rows_v, tbl_sh, acc_sh, gsem, ssem):
    cid = lax.axis_index("c")
    sid = lax.axis_index("s")
    wid = sid * NC + cid
    base, cnt = _worker_span(wid)
    s = pl.ds(sid * STRIPE, STRIPE)
    stages = (
        (zeros_hbm.at[s], acc_sh.at[s]),
        (table_hbm.at[s], tbl_sh.at[s]),
        (src_hbm.at[pl.ds(base, MAXC)], src_v),
        (dst_hbm.at[pl.ds(base, MAXC)], dst_v),
    )
    for i, (a, b_) in enumerate(stages):
        pltpu.async_copy(a, b_, gsem.at[i])
    for i, (a, b_) in enumerate(stages):
        pltpu.make_async_copy(a, b_, gsem.at[i]).wait()
    plsc.subcore_barrier()
    def step(c, _):
        b = lax.rem(c, KB)
        p = c - PD
        bp = lax.rem(p + KB, KB)
        @pl.when(c < cnt)
        def _g():
            @pl.when(c >= KB)
            def _w():
                pltpu.make_async_copy(rows_v.at[b], acc_sh.at[src_v.at[0]],
                                      ssem.at[b]).wait()
            pltpu.async_copy(tbl_sh.at[src_v.at[c]], rows_v.at[b], gsem.at[b])
        @pl.when(c >= PD)
        def _s():
            pltpu.make_async_copy(tbl_sh.at[src_v.at[0]], rows_v.at[bp],
                                  gsem.at[bp]).wait()
            pltpu.async_copy(rows_v.at[bp], acc_sh.at[dst_v.at[p]],
                             ssem.at[bp], add=True)
        return _
    lax.fori_loop(0, cnt + PD, step, 0)
    def drain(j, _):
        @pl.when(j < cnt)
        def _w():
            pltpu.make_async_copy(rows_v.at[j], acc_sh.at[src_v.at[0]],
                                  ssem.at[j]).wait()
        return _
    lax.fori_loop(0, KB, drain, 0)
    plsc.subcore_barrier()
    pltpu.sync_copy(acc_sh.at[s], out_hbm.at[cid, s])


def _sc_degrees(src_r, dst_r, zeros):
    return pl.kernel(
        _deg_body,
        out_type=jax.ShapeDtypeStruct((NC, 2, NP, F), jnp.float32),
        mesh=_sc_mesh(),
        compiler_params=_SC_PARAMS,
        scratch_types=[
            pltpu.VMEM((MAXC, CH), jnp.int32),
            pltpu.VMEM((MAXC, CH), jnp.int32),
            pltpu.VMEM((CH, F), jnp.float32),
            pltpu.VMEM_SHARED((NP, F), jnp.float32),
            pltpu.VMEM_SHARED((NP, F), jnp.float32),
            pltpu.SemaphoreType.DMA,
        ],
    )(src_r, dst_r, zeros)


def _sc_aggregate(table, src_r, dst_r, zeros):
    return pl.kernel(
        _agg_body,
        out_type=jax.ShapeDtypeStruct((NC, NP, F), jnp.float32),
        mesh=_sc_mesh(),
        compiler_params=_SC_PARAMS,
        scratch_types=[
            pltpu.VMEM((MAXC, CH), jnp.int32),
            pltpu.VMEM((MAXC, CH), jnp.int32),
            pltpu.VMEM((KB, CH, F), jnp.float32),
            pltpu.VMEM_SHARED((NP, F), jnp.float32),
            pltpu.VMEM_SHARED((NP, F), jnp.float32),
            pltpu.SemaphoreType.DMA((KB,)),
            pltpu.SemaphoreType.DMA((KB,)),
        ],
    )(table, src_r, dst_r, zeros)


def _prep1_body(x_ref, w_ref, deg_ref, t_ref, rin_ref, rout_ref):
    dout = deg_ref[0, 0] + deg_ref[1, 0]
    din = deg_ref[0, 1] + deg_ref[1, 1]
    rout = lax.rsqrt(jnp.maximum(dout, 1.0))
    rin = lax.rsqrt(jnp.maximum(din, 1.0))
    rout_ref[...] = rout
    rin_ref[...] = rin
    xall = x_ref[...].reshape(XR, 8, D_IN)
    w = w_ref[...]
    for c in range(8):
        xc = xall[:, c, :] * rout[0:XR, c * F:c * F + 1]
        t_ref[0:XR, c * F:(c + 1) * F] = jnp.dot(
            xc, w, preferred_element_type=jnp.float32)


def _tc_prep1(x, w1p, deg):
    return pl.pallas_call(
        _prep1_body,
        out_shape=(jax.ShapeDtypeStruct((VR, 128), jnp.float32),
                   jax.ShapeDtypeStruct((VR, 128), jnp.float32),
                   jax.ShapeDtypeStruct((VR, 128), jnp.float32)),
    )(x, w1p, deg)


def _prep_body(part_ref, rin_ref, rout_ref, b_ref, w_ref, t_ref):
    agg = part_ref[0] + part_ref[1]
    h = jnp.maximum(agg * rin_ref[...] + b_ref[...], 0.0)
    t_ref[...] = jnp.dot(h * rout_ref[...], w_ref[...],
                         preferred_element_type=jnp.float32)


def _tc_prep(part, rin, rout, b128, wblk):
    return pl.pallas_call(
        _prep_body,
        out_shape=jax.ShapeDtypeStruct((VR, 128), jnp.float32),
    )(part, rin, rout, b128, wblk)


def _final_body(part_ref, rin_ref, b_ref, l1w_ref, l1b_ref, l2w_ref, l2b_ref,
                l3w_ref, l3b_ref, out_ref):
    agg = part_ref[0] + part_ref[1]
    h = jnp.maximum(agg * rin_ref[...] + b_ref[...], 0.0)
    rows = lax.broadcasted_iota(jnp.int32, (VR, 128), 0)
    h = jnp.where(rows < XR, h, 0.0)
    s = jnp.sum(h, axis=0, keepdims=True)
    fold = jnp.where(lax.broadcasted_iota(jnp.int32, (128, F), 0) % F
                     == lax.broadcasted_iota(jnp.int32, (128, F), 1),
                     1.0, 0.0)
    hg = jnp.dot(s, fold, preferred_element_type=jnp.float32)[:, :H] / N
    hg = jnp.maximum(jnp.dot(hg, l1w_ref[...],
                             preferred_element_type=jnp.float32)
                     + l1b_ref[...], 0.0)
    hg = jnp.maximum(jnp.dot(hg, l2w_ref[...],
                             preferred_element_type=jnp.float32)
                     + l2b_ref[...], 0.0)
    r = jnp.dot(hg, l3w_ref[...], preferred_element_type=jnp.float32)
    out_ref[...] = r + l3b_ref[...]


def _tc_final(part, rin, b3_128, l1w, l1b, l2w, l2b, l3w, l3b):
    return pl.pallas_call(
        _final_body,
        out_shape=jax.ShapeDtypeStruct((1, 1), jnp.float32),
    )(part, rin, b3_128, l1w, l1b, l2w, l2b, l3w, l3b)


def kernel(x, edge_index, W1, b1, W2, b2, W3, b3, L1W, L1b, L2W, L2b, L3W, L3b):
    f32 = jnp.float32
    src_r = edge_index[0].reshape(NCHT, CH)
    dst_r = edge_index[1].reshape(NCHT, CH)
    zeros = jnp.zeros((NP, F), f32)
    eye8 = np.eye(8, dtype=np.float32)

    def padw(w):
        return jnp.zeros((F, F), f32).at[:w.shape[0], :w.shape[1]].set(w)

    def tile_b(b):
        return jnp.tile(jnp.zeros((F,), f32).at[:H].set(b), 8).reshape(1, 128)

    w1p = jnp.zeros((D_IN, F), f32).at[:, :H].set(W1)
    w2b = jnp.kron(eye8, padw(W2))
    w3b = jnp.kron(eye8, padw(W3))
    b1t = tile_b(b1)
    b2t = tile_b(b2)
    b3t = tile_b(b3)

    def vw(a):
        return a.reshape(a.shape[:-2] + (VR, 128))

    def rw(a):
        return a.reshape(NP, F)

    deg = _sc_degrees(src_r, dst_r, zeros)
    t1, rin, rout = _tc_prep1(x, w1p, vw(deg))
    p1 = _sc_aggregate(rw(t1), src_r, dst_r, zeros)
    t2 = _tc_prep(vw(p1), rin, rout, b1t, w2b)
    p2 = _sc_aggregate(rw(t2), src_r, dst_r, zeros)
    t3 = _tc_prep(vw(p2), rin, rout, b2t, w3b)
    p3 = _sc_aggregate(rw(t3), src_r, dst_r, zeros)
    return _tc_final(vw(p3), rin, b3t, L1W, L1b.reshape(1, H),
                     L2W, L2b.reshape(1, H), L3W, L3b.reshape(1, 1))


# speedup vs baseline: 1.0258x; 1.0258x over previous
import functools

import jax
import jax.numpy as jnp
import numpy as np
from jax import lax
from jax.experimental import pallas as pl
from jax.experimental.pallas import tpu as pltpu
from jax.experimental.pallas import tpu_sc as plsc

N = 10000
E = 320000
D_IN = 128
H = 12
F = 16
NC = 2
NS = 16
NW = NC * NS
CH = 128
NCHT = E // CH
MAXC = NCHT // NW + 1
KB = 8
PD = 4
NP = 10240
STRIPE = NP // NS
VR = NP // 8
XR = N // 8


def _sc_mesh():
    return plsc.VectorSubcoreMesh(core_axis_name="c", subcore_axis_name="s",
                                  num_cores=NC, num_subcores=NS)


_SC_PARAMS = pltpu.CompilerParams(use_tc_tiling_on_sc=False)


def _worker_span(wid):
    base = (wid * NCHT) // NW
    cnt = (((wid + 1) * NCHT) // NW) - base
    return base, cnt


def _deg_body(src_hbm, dst_hbm, zeros_hbm, out_hbm,
              src_v, dst_v, ones_v, dout_sh, din_sh, sem):
    cid = lax.axis_index("c")
    sid = lax.axis_index("s")
    wid = sid * NC + cid
    base, cnt = _worker_span(wid)
    zrows = zeros_hbm.at[pl.ds(sid * STRIPE, STRIPE)]
    stages = (
        (zrows, dout_sh.at[pl.ds(sid * STRIPE, STRIPE)]),
        (zrows, din_sh.at[pl.ds(sid * STRIPE, STRIPE)]),
        (src_hbm.at[pl.ds(base, MAXC)], src_v),
        (dst_hbm.at[pl.ds(base, MAXC)], dst_v),
    )
    for a, b_ in stages:
        pltpu.async_copy(a, b_, sem)
    for a, b_ in stages:
        pltpu.make_async_copy(a, b_, sem).wait()
    def fill(i, _):
        ones_v[i, :] = jnp.full((F,), 1.0, jnp.float32)
        return _
    lax.fori_loop(0, CH, fill, 0)
    plsc.subcore_barrier()
    def chunk(c, _):
        pltpu.async_copy(ones_v, dout_sh.at[src_v.at[c]], sem, add=True)
        pltpu.async_copy(ones_v, din_sh.at[dst_v.at[c]], sem, add=True)
        return _
    lax.fori_loop(0, cnt, chunk, 0)
    def drain(c, _):
        pltpu.make_async_copy(ones_v, dout_sh.at[src_v.at[0]], sem).wait()
        pltpu.make_async_copy(ones_v, din_sh.at[dst_v.at[0]], sem).wait()
        return _
    lax.fori_loop(0, cnt, drain, 0)
    plsc.subcore_barrier()
    s = pl.ds(sid * STRIPE, STRIPE)
    pltpu.sync_copy(dout_sh.at[s], out_hbm.at[cid, 0, s])
    pltpu.sync_copy(din_sh.at[s], out_hbm.at[cid, 1, s])


def _agg_body(table_hbm, src_hbm, dst_hbm, zeros_hbm, out_hbm,
              src_v, dst_v, rows_v, tbl_sh, acc_sh, gsem, ssem):
    cid = lax.axis_index("c")
    sid = lax.axis_index("s")
    wid = sid * NC + cid
    base, cnt = _worker_span(wid)
    s = pl.ds(sid * STRIPE, STRIPE)
    stages = (
        (zeros_hbm.at[s], acc_sh.at[s]),
        (table_hbm.at[s], tbl_sh.at[s]),
        (src_hbm.at[pl.ds(base, MAXC)], src_v),
        (dst_hbm.at[pl.ds(base, MAXC)], dst_v),
    )
    for i, (a, b_) in enumerate(stages):
        pltpu.async_copy(a, b_, gsem.at[i])
    for i, (a, b_) in enumerate(stages):
        pltpu.make_async_copy(a, b_, gsem.at[i]).wait()
    plsc.subcore_barrier()
    def step(c, _):
        b = lax.rem(c, KB)
        p = c - PD
        bp = lax.rem(p + KB, KB)
        @pl.when(c < cnt)
        def _g():
            @pl.when(c >= KB)
            def _w():
                pltpu.make_async_copy(rows_v.at[b], acc_sh.at[src_v.at[0]],
                                      ssem.at[b]).wait()
            pltpu.async_copy(tbl_sh.at[src_v.at[c]], rows_v.at[b], gsem.at[b])
        @pl.when(c >= PD)
        def _s():
            pltpu.make_async_copy(tbl_sh.at[src_v.at[0]], rows_v.at[bp],
                                  gsem.at[bp]).wait()
            pltpu.async_copy(rows_v.at[bp], acc_sh.at[dst_v.at[p]],
                             ssem.at[bp], add=True)
        return _
    lax.fori_loop(0, cnt + PD, step, 0)
    def drain(j, _):
        @pl.when(j < cnt)
        def _w():
            pltpu.make_async_copy(rows_v.at[j], acc_sh.at[src_v.at[0]],
                                  ssem.at[j]).wait()
        return _
    lax.fori_loop(0, KB, drain, 0)
    plsc.subcore_barrier()
    pltpu.sync_copy(acc_sh.at[s], out_hbm.at[cid, s])


def _sc_degrees(src_r, dst_r, zeros):
    return pl.kernel(
        _deg_body,
        out_type=jax.ShapeDtypeStruct((NC, 2, NP, F), jnp.float32),
        mesh=_sc_mesh(),
        compiler_params=_SC_PARAMS,
        scratch_types=[
            pltpu.VMEM((MAXC, CH), jnp.int32),
            pltpu.VMEM((MAXC, CH), jnp.int32),
            pltpu.VMEM((CH, F), jnp.float32),
            pltpu.VMEM_SHARED((NP, F), jnp.float32),
            pltpu.VMEM_SHARED((NP, F), jnp.float32),
            pltpu.SemaphoreType.DMA,
        ],
    )(src_r, dst_r, zeros)


def _sc_aggregate(table, src_r, dst_r, zeros):
    return pl.kernel(
        _agg_body,
        out_type=jax.ShapeDtypeStruct((NC, NP, F), jnp.float32),
        mesh=_sc_mesh(),
        compiler_params=_SC_PARAMS,
        scratch_types=[
            pltpu.VMEM((MAXC, CH), jnp.int32),
            pltpu.VMEM((MAXC, CH), jnp.int32),
            pltpu.VMEM((KB, CH, F), jnp.float32),
            pltpu.VMEM_SHARED((NP, F), jnp.float32),
            pltpu.VMEM_SHARED((NP, F), jnp.float32),
            pltpu.SemaphoreType.DMA((KB,)),
            pltpu.SemaphoreType.DMA((KB,)),
        ],
    )(table, src_r, dst_r, zeros)


def _prep1_body(x_ref, w_ref, deg_ref, t_ref, rin_ref, rout_ref):
    dout = deg_ref[0, 0] + deg_ref[1, 0]
    din = deg_ref[0, 1] + deg_ref[1, 1]
    rout = lax.rsqrt(jnp.maximum(dout, 1.0))
    rin = lax.rsqrt(jnp.maximum(din, 1.0))
    rout_ref[...] = rout
    rin_ref[...] = rin
    xall = x_ref[...].reshape(XR, 8, D_IN)
    w = w_ref[...]
    for c in range(8):
        xc = xall[:, c, :]
        yc = jnp.dot(xc, w, preferred_element_type=jnp.float32)
        t_ref[0:XR, c * F:(c + 1) * F] = yc * rout[0:XR, c * F:(c + 1) * F]


def _tc_prep1(x, w1p, deg):
    return pl.pallas_call(
        _prep1_body,
        out_shape=(jax.ShapeDtypeStruct((VR, 128), jnp.float32),
                   jax.ShapeDtypeStruct((VR, 128), jnp.float32),
                   jax.ShapeDtypeStruct((VR, 128), jnp.float32)),
    )(x, w1p, deg)


def _prep_body(part_ref, rin_ref, rout_ref, b_ref, w_ref, t_ref):
    agg = part_ref[0] + part_ref[1]
    h = jnp.maximum(agg * rin_ref[...] + b_ref[...], 0.0)
    y = jnp.dot(h, w_ref[...], preferred_element_type=jnp.float32)
    t_ref[...] = y * rout_ref[...]


def _tc_prep(part, rin, rout, b128, wblk):
    return pl.pallas_call(
        _prep_body,
        out_shape=jax.ShapeDtypeStruct((VR, 128), jnp.float32),
    )(part, rin, rout, b128, wblk)


def _final_body(part_ref, rin_ref, b_ref, l1w_ref, l1b_ref, l2w_ref, l2b_ref,
                l3w_ref, l3b_ref, out_ref):
    agg = part_ref[0] + part_ref[1]
    h = jnp.maximum(agg * rin_ref[...] + b_ref[...], 0.0)
    rows = lax.broadcasted_iota(jnp.int32, (VR, 128), 0)
    h = jnp.where(rows < XR, h, 0.0)
    s = jnp.sum(h, axis=0, keepdims=True)
    fold = jnp.where(lax.broadcasted_iota(jnp.int32, (128, F), 0) % F
                     == lax.broadcasted_iota(jnp.int32, (128, F), 1),
                     1.0, 0.0)
    hg = jnp.dot(s, fold, preferred_element_type=jnp.float32)[:, :H] / N
    hg = jnp.maximum(jnp.dot(hg, l1w_ref[...],
                             preferred_element_type=jnp.float32)
                     + l1b_ref[...], 0.0)
    hg = jnp.maximum(jnp.dot(hg, l2w_ref[...],
                             preferred_element_type=jnp.float32)
                     + l2b_ref[...], 0.0)
    r = jnp.dot(hg, l3w_ref[...], preferred_element_type=jnp.float32)
    out_ref[...] = r + l3b_ref[...]


def _tc_final(part, rin, b3_128, l1w, l1b, l2w, l2b, l3w, l3b):
    return pl.pallas_call(
        _final_body,
        out_shape=jax.ShapeDtypeStruct((1, 1), jnp.float32),
    )(part, rin, b3_128, l1w, l1b, l2w, l2b, l3w, l3b)


def kernel(x, edge_index, W1, b1, W2, b2, W3, b3, L1W, L1b, L2W, L2b, L3W, L3b):
    f32 = jnp.float32
    src_r = edge_index[0].reshape(NCHT, CH)
    dst_r = edge_index[1].reshape(NCHT, CH)
    zeros = jnp.zeros((NP, F), f32)
    eye8 = np.eye(8, dtype=np.float32)

    def padw(w):
        return jnp.zeros((F, F), f32).at[:w.shape[0], :w.shape[1]].set(w)

    def tile_b(b):
        return jnp.tile(jnp.zeros((F,), f32).at[:H].set(b), 8).reshape(1, 128)

    w1p = jnp.zeros((D_IN, F), f32).at[:, :H].set(W1)
    w2b = jnp.kron(eye8, padw(W2))
    w3b = jnp.kron(eye8, padw(W3))
    b1t = tile_b(b1)
    b2t = tile_b(b2)
    b3t = tile_b(b3)

    def vw(a):
        return a.reshape(a.shape[:-2] + (VR, 128))

    def rw(a):
        return a.reshape(NP, F)

    deg = _sc_degrees(src_r, dst_r, zeros)
    t1, rin, rout = _tc_prep1(x, w1p, vw(deg))
    p1 = _sc_aggregate(rw(t1), src_r, dst_r, zeros)
    t2 = _tc_prep(vw(p1), rin, rout, b1t, w2b)
    p2 = _sc_aggregate(rw(t2), src_r, dst_r, zeros)
    t3 = _tc_prep(vw(p2), rin, rout, b2t, w3b)
    p3 = _sc_aggregate(rw(t3), src_r, dst_r, zeros)
    return _tc_final(vw(p3), rin, b3t, L1W, L1b.reshape(1, H),
                     L2W, L2b.reshape(1, H), L3W, L3b.reshape(1, 1))
